# baseline (speedup 1.0000x reference)
import functools

import jax
import jax.numpy as jnp
from jax import lax
from jax.experimental import pallas as pl
from jax.experimental.pallas import tpu as pltpu
from jax.experimental.pallas import tpu_sc as plsc

_INFO = plsc.get_sparse_core_info()
_NC = _INFO.num_cores
_NS = _INFO.num_subcores
_NW = _NC * _NS

_BATCH = 4
_T = 2048
_B = _BATCH * _T
_F = 1728
_V = 23236
_NJT = _F // 8
_JPW = (_NJT + _NW - 1) // _NW

_CW = 2560
_CWB = 2624
_VT = 9 * _CW
_TAILW = 256


def _make_kernel():
    mesh = plsc.VectorSubcoreMesh(core_axis_name="c", subcore_axis_name="s")

    @functools.partial(
        pl.kernel,
        out_type=jax.ShapeDtypeStruct((_BATCH, _F, _T), jnp.float32),
        mesh=mesh,
        compiler_params=pltpu.CompilerParams(
            use_tc_tiling_on_sc=True, needs_layout_passes=False),
        scratch_types=[
            pltpu.VMEM((_B,), jnp.int32),
            pltpu.VMEM((_B,), jnp.int32),
            pltpu.VMEM((16,), jnp.int32),
            pltpu.VMEM((2, 8, _CWB), jnp.float32),
            pltpu.VMEM((8, _TAILW + 64), jnp.float32),
            pltpu.VMEM((8, _B), jnp.float32),
            pltpu.VMEM((16,), jnp.int32),
            pltpu.SemaphoreType.DMA,
            pltpu.SemaphoreType.DMA,
            pltpu.SemaphoreType.DMA,
            pltpu.SemaphoreType.DMA,
        ],
    )
    def lane_kernel(wt_hbm, wtail_hbm, sidx_hbm, stok_hbm, bnd_hbm, out_hbm,
                    sidxv, stokv, idx8, tb, tbl2, oall, bnd, g0, g1, g2, psem):
        wid = lax.axis_index("s") * _NC + lax.axis_index("c")

        pltpu.sync_copy(sidx_hbm, sidxv)
        pltpu.sync_copy(stok_hbm, stokv)
        pltpu.sync_copy(bnd_hbm, bnd)

        gsem = (g0, g1)
        rows = [jnp.full((16,), s, jnp.int32) for s in range(8)]
        lane = lax.iota(jnp.int32, 16)

        def chunk_copy(k, par):
            r8 = idx8.at[pl.ds(0, 8)]
            return pltpu.make_async_copy(
                wt_hbm.at[r8, pl.ds(k * _CW, _CW)],
                tb.at[par].at[:, pl.ds(0, _CW)], gsem[par],
            )

        def process(k, buf, base):
            bv = bnd[pl.ds(0, 16)]
            lo_pos = jnp.sum(jnp.where(lane == k, bv, 0))
            hi_pos = jnp.sum(jnp.where(lane == k + 1, bv, 0))

            def qstep(q, carry):
                for hh in range(4):
                    p = q * 64 + hh * 16
                    sv = sidxv[pl.ds(p, 16)]
                    tv = stokv[pl.ds(p, 16)]
                    pos = p + lane
                    m = (pos >= lo_pos) & (pos < hi_pos)
                    col = sv - base
                    for s in range(8):
                        vals = plsc.load_gather(buf, [rows[s], col], mask=m)
                        plsc.store_scatter(oall, [rows[s], tv], vals, mask=m)
                return carry

            lax.fori_loop(lo_pos // 64, (hi_pos + 63) // 64, qstep, 0)

        def drain_out():
            for b in range(_BATCH):
                pltpu.make_async_copy(
                    oall.at[:, pl.ds(b * _T, _T)],
                    out_hbm.at[b, pl.ds(0, 8)], psem,
                ).wait()

        for i in range(_JPW):
            jt = wid + _NW * i

            @pl.when(jt < _NJT)
            def _():
                idx8[...] = jt * 8 + lax.iota(jnp.int32, 16)
                ht = pltpu.async_copy(
                    wtail_hbm.at[idx8.at[pl.ds(0, 8)]],
                    tbl2.at[:, pl.ds(0, _TAILW)], g2,
                )
                chunk_copy(0, 0).start()
                chunk_copy(1, 1).start()
                if i > 0:
                    drain_out()

                def pair(k2, carry):
                    k = 2 * k2
                    chunk_copy(k, 0).wait()
                    process(k, tb.at[0], k * _CW)

                    @pl.when(k + 2 <= 8)
                    def _():
                        chunk_copy(k + 2, 0).start()

                    chunk_copy(k + 1, 1).wait()
                    process(k + 1, tb.at[1], (k + 1) * _CW)

                    @pl.when(k + 3 <= 8)
                    def _():
                        chunk_copy(k + 3, 1).start()

                    return carry

                lax.fori_loop(0, 4, pair, 0)

                chunk_copy(8, 0).wait()
                process(8, tb.at[0], 8 * _CW)
                ht.wait()
                process(9, tbl2, _VT)

                for b in range(_BATCH):
                    pltpu.async_copy(
                        oall.at[:, pl.ds(b * _T, _T)],
                        out_hbm.at[b, pl.ds(jt * 8, 8)], psem,
                    )

        drain_out()

    return lane_kernel


@jax.jit
def kernel(input_ids, weight):
    wt = weight.T
    wtail = jnp.pad(wt[:, _VT:], ((0, 0), (0, _TAILW - (_V - _VT))))
    idx = input_ids.reshape(-1).astype(jnp.int32)
    sidx, stok = lax.sort_key_val(idx, lax.iota(jnp.int32, _B))
    edges = jnp.minimum(jnp.arange(16, dtype=jnp.int32) * _CW, _V)
    bnd = jnp.searchsorted(sidx, edges).astype(jnp.int32)
    out3 = _make_kernel()(wt, wtail, sidx, stok, bnd)
    return jnp.transpose(out3, (0, 2, 1))


# speedup vs baseline: 1.0322x; 1.0322x over previous
import functools

import jax
import jax.numpy as jnp
from jax import lax
from jax.experimental import pallas as pl
from jax.experimental.pallas import tpu as pltpu
from jax.experimental.pallas import tpu_sc as plsc

_INFO = plsc.get_sparse_core_info()
_NC = _INFO.num_cores
_NS = _INFO.num_subcores
_NW = _NC * _NS

_BATCH = 4
_T = 2048
_B = _BATCH * _T
_F = 1728
_V = 23236
_NJT = _F // 8
_JPW = (_NJT + _NW - 1) // _NW

_CW = 2560
_CWB = 2624
_VT = 9 * _CW
_TAILW = 256


def _make_kernel():
    mesh = plsc.VectorSubcoreMesh(core_axis_name="c", subcore_axis_name="s")

    @functools.partial(
        pl.kernel,
        out_type=jax.ShapeDtypeStruct((_BATCH, _F, _T), jnp.float32),
        mesh=mesh,
        compiler_params=pltpu.CompilerParams(
            use_tc_tiling_on_sc=True, needs_layout_passes=False),
        scratch_types=[
            pltpu.VMEM((_B,), jnp.int32),
            pltpu.VMEM((_B,), jnp.int32),
            pltpu.VMEM((16,), jnp.int32),
            pltpu.VMEM((2, 8, _CWB), jnp.float32),
            pltpu.VMEM((8, _TAILW + 64), jnp.float32),
            pltpu.VMEM((8, _B), jnp.float32),
            pltpu.VMEM((16,), jnp.int32),
            pltpu.SemaphoreType.DMA,
            pltpu.SemaphoreType.DMA,
            pltpu.SemaphoreType.DMA,
            pltpu.SemaphoreType.DMA,
        ],
    )
    def lane_kernel(wt_hbm, wtail_hbm, sidx_hbm, stok_hbm, bnd_hbm, out_hbm,
                    sidxv, stokv, idx8, tb, tbl2, oall, bnd, g0, g1, g2, psem):
        wid = lax.axis_index("s") * _NC + lax.axis_index("c")

        pltpu.sync_copy(sidx_hbm, sidxv)
        pltpu.sync_copy(stok_hbm, stokv)
        pltpu.sync_copy(bnd_hbm, bnd)

        gsem = (g0, g1)
        rows = [jnp.full((16,), s, jnp.int32) for s in range(8)]
        lane = lax.iota(jnp.int32, 16)

        def chunk_copy(k, par):
            r8 = idx8.at[pl.ds(0, 8)]
            return pltpu.make_async_copy(
                wt_hbm.at[r8, pl.ds(k * _CW, _CW)],
                tb.at[par].at[:, pl.ds(0, _CW)], gsem[par],
            )

        def process(k, buf, base):
            bv = bnd[pl.ds(0, 16)]
            lo_pos = jnp.sum(jnp.where(lane == k, bv, 0))
            hi_pos = jnp.sum(jnp.where(lane == k + 1, bv, 0))

            def qstep(q, carry):
                for hh in range(2):
                    p = q * 32 + hh * 16
                    sv = sidxv[pl.ds(p, 16)]
                    tv = stokv[pl.ds(p, 16)]
                    pos = p + lane
                    m = (pos >= lo_pos) & (pos < hi_pos)
                    col = sv - base
                    for s in range(8):
                        vals = plsc.load_gather(buf, [rows[s], col], mask=m)
                        plsc.store_scatter(oall, [rows[s], tv], vals, mask=m)
                return carry

            lax.fori_loop(lo_pos // 32, (hi_pos + 31) // 32, qstep, 0)

        def drain_out():
            for b in range(_BATCH):
                pltpu.make_async_copy(
                    oall.at[:, pl.ds(b * _T, _T)],
                    out_hbm.at[b, pl.ds(0, 8)], psem,
                ).wait()

        idx8[...] = wid * 8 + lax.iota(jnp.int32, 16)
        chunk_copy(0, 0).start()
        chunk_copy(1, 1).start()

        for i in range(_JPW):
            jt = wid + _NW * i

            @pl.when(jt < _NJT)
            def _():
                ht = pltpu.async_copy(
                    wtail_hbm.at[idx8.at[pl.ds(0, 8)]],
                    tbl2.at[:, pl.ds(0, _TAILW)], g2,
                )
                if i > 0:
                    drain_out()

                def pair(k2, carry):
                    k = 2 * k2
                    chunk_copy(k, 0).wait()
                    process(k, tb.at[0], k * _CW)

                    @pl.when(k + 2 <= 8)
                    def _():
                        chunk_copy(k + 2, 0).start()

                    chunk_copy(k + 1, 1).wait()
                    process(k + 1, tb.at[1], (k + 1) * _CW)

                    @pl.when(k + 3 <= 8)
                    def _():
                        chunk_copy(k + 3, 1).start()

                    return carry

                lax.fori_loop(0, 4, pair, 0)

                chunk_copy(8, 0).wait()
                process(8, tb.at[0], 8 * _CW)
                ht.wait()
                if i + 1 < _JPW:
                    nxt = jt + _NW

                    @pl.when(nxt < _NJT)
                    def _():
                        idx8[...] = nxt * 8 + lax.iota(jnp.int32, 16)
                        chunk_copy(0, 0).start()
                        chunk_copy(1, 1).start()

                process(9, tbl2, _VT)

                for b in range(_BATCH):
                    pltpu.async_copy(
                        oall.at[:, pl.ds(b * _T, _T)],
                        out_hbm.at[b, pl.ds(jt * 8, 8)], psem,
                    )

        drain_out()

    return lane_kernel


@jax.jit
def kernel(input_ids, weight):
    wt = weight.T
    wtail = jnp.pad(wt[:, _VT:], ((0, 0), (0, _TAILW - (_V - _VT))))
    idx = input_ids.reshape(-1).astype(jnp.int32)
    sidx, stok = lax.sort_key_val(idx, lax.iota(jnp.int32, _B))
    edges = jnp.minimum(jnp.arange(16, dtype=jnp.int32) * _CW, _V)
    bnd = jnp.searchsorted(sidx, edges).astype(jnp.int32)
    out3 = _make_kernel()(wt, wtail, sidx, stok, bnd)
    return jnp.transpose(out3, (0, 2, 1))
